# Initial kernel scaffold; baseline (speedup 1.0000x reference)
import jax
import jax.numpy as jnp
from jax.experimental import pallas as pl


def kernel(x, W, b):
    raise NotImplementedError("write your pallas kernel here")


# speedup vs baseline: 2.7680x; 2.7680x over previous
import jax
import jax.numpy as jnp
from jax.experimental import pallas as pl

_KS = 25
_NBITS = 12


def _first_argmax(a, fill):
    m = jnp.max(a, axis=-1, keepdims=True)
    idx = jax.lax.broadcasted_iota(jnp.int32, a.shape, 1)
    only = jnp.where(a == m, idx, fill)
    return jnp.min(only, axis=-1, keepdims=True)


def _block_kernel(x_ref, ac_ref, w_ref, b_ref, out_ref, trend_ref, agg_ref):
    x = x_ref[0]
    ac = ac_ref[0]
    C, T = x.shape

    acc = x
    for j in range(1, _KS // 2 + 1):
        z = jnp.zeros((C, j), x.dtype)
        acc = acc + jnp.concatenate([z, x[:, : T - j]], axis=1)
        acc = acc + jnp.concatenate([x[:, j:], z], axis=1)
    trend = acc * (1.0 / _KS)
    s = x - trend

    lane = jax.lax.broadcasted_iota(jnp.int32, ac.shape, 1)
    neginf = jnp.float32(-jnp.inf)
    cand = jnp.where(lane == 0, neginf, ac)
    m1 = _first_argmax(cand, T)
    cand = jnp.where(lane == m1, neginf, cand)
    m2 = _first_argmax(cand, T)

    def shifted(L):
        y = s
        for bit in range(_NBITS):
            amt = 1 << bit
            rolled = jnp.concatenate([y[:, T - amt:], y[:, : T - amt]], axis=1)
            y = jnp.where(((L >> bit) & 1) == 1, rolled, y)
        return jnp.where(lane >= L, y, 0.0)

    agg = (s + shifted(m1) + shifted(m2)) * (1.0 / 3.0)

    st = agg + trend
    out = jax.lax.dot_general(
        w_ref[...], st, (((1,), (0,)), ((), ())),
        preferred_element_type=jnp.float32,
        precision=jax.lax.Precision.HIGHEST) + b_ref[...]

    out_ref[0] = out
    trend_ref[0] = trend
    agg_ref[0] = agg


def kernel(x, W, b):
    B, C, T = x.shape

    pad = _KS // 2
    xp = jnp.pad(x, ((0, 0), (0, 0), (pad, pad)))
    cs = jnp.cumsum(xp, axis=-1)
    zero = jnp.zeros(cs.shape[:-1] + (1,), cs.dtype)
    cs = jnp.concatenate([zero, cs], axis=-1)
    trend_e = (cs[..., _KS:] - cs[..., :-_KS]) / _KS
    s_e = x - trend_e
    fx = jnp.fft.rfft(s_e, axis=-1)
    ac = jnp.fft.irfft(fx * jnp.conj(fx), n=T, axis=-1)

    bs = pl.BlockSpec((1, C, T), lambda i: (i, 0, 0))
    out, trend, agg = pl.pallas_call(
        _block_kernel,
        grid=(B,),
        in_specs=[
            bs, bs,
            pl.BlockSpec((C, C), lambda i: (0, 0)),
            pl.BlockSpec((C, 1), lambda i: (0, 0)),
        ],
        out_specs=[bs, bs, bs],
        out_shape=[jax.ShapeDtypeStruct((B, C, T), jnp.float32)] * 3,
    )(x, ac, W, b.reshape(C, 1))
    return out, trend, agg


# speedup vs baseline: 4.7912x; 1.7309x over previous
import jax
import jax.numpy as jnp
from jax.experimental import pallas as pl
from jax.experimental.pallas import tpu as pltpu

_KS = 25
_NBITS = 12
_CH = 128


def _decomp_kernel(x_ref, trend_ref, se_ref, scan_ref):
    x = x_ref[0]
    C, T = x.shape
    npad_l = _KS // 2
    n_in = T + 2 * npad_l
    nch = (n_in + _CH - 1) // _CH
    npad_r = nch * _CH - T - npad_l

    zl = jnp.zeros((C, npad_l), jnp.float32)
    zr = jnp.zeros((C, npad_r), jnp.float32)
    xp = jnp.concatenate([zl, x, zr], axis=1)

    cols = [xp[:, c * _CH:(c + 1) * _CH].T for c in range(nch)]
    scan_ref[...] = jnp.concatenate(cols, axis=1)

    def body(r, carry):
        nxt = carry + scan_ref[pl.ds(r, 1), :]
        scan_ref[pl.ds(r, 1), :] = nxt
        return nxt

    totals = jax.lax.fori_loop(1, _CH, body, scan_ref[pl.ds(0, 1), :])

    offs = [jnp.zeros((1, C), jnp.float32)]
    for c in range(1, nch):
        offs.append(offs[-1] + totals[:, (c - 1) * C:c * C])

    cs_cols = []
    for c in range(nch):
        blk = scan_ref[:, c * C:(c + 1) * C] + offs[c]
        cs_cols.append(blk.T)
    cs = jnp.concatenate(cs_cols, axis=1)

    hi = cs[:, _KS - 1:_KS - 1 + T]
    lo = jnp.concatenate([jnp.zeros((C, 1), jnp.float32), cs[:, :T - 1]],
                         axis=1)
    trend = (hi - lo) / jnp.float32(_KS)
    trend_ref[0] = trend
    se_ref[0] = x - trend


def _first_argmax(a, fill):
    m = jnp.max(a, axis=-1, keepdims=True)
    idx = jax.lax.broadcasted_iota(jnp.int32, a.shape, 1)
    only = jnp.where(a == m, idx, fill)
    return jnp.min(only, axis=-1, keepdims=True)


def _agg_kernel(ac_ref, s_ref, trend_ref, w_ref, b_ref, out_ref, agg_ref):
    ac = ac_ref[0]
    s = s_ref[0]
    trend = trend_ref[0]
    C, T = s.shape

    lane = jax.lax.broadcasted_iota(jnp.int32, ac.shape, 1)
    neginf = jnp.float32(-jnp.inf)
    cand = jnp.where(lane == 0, neginf, ac)
    m1 = _first_argmax(cand, T)
    cand = jnp.where(lane == m1, neginf, cand)
    m2 = _first_argmax(cand, T)

    def shifted(L):
        y = s
        for bit in range(_NBITS):
            amt = 1 << bit
            rolled = jnp.concatenate([y[:, T - amt:], y[:, : T - amt]], axis=1)
            y = jnp.where(((L >> bit) & 1) == 1, rolled, y)
        return jnp.where(lane >= L, y, 0.0)

    agg = (s + shifted(m1) + shifted(m2)) * (1.0 / 3.0)

    out = jax.lax.dot_general(
        w_ref[...], agg + trend, (((1,), (0,)), ((), ())),
        preferred_element_type=jnp.float32,
        precision=jax.lax.Precision.HIGHEST) + b_ref[...]

    out_ref[0] = out
    agg_ref[0] = agg


def kernel(x, W, b):
    B, C, T = x.shape
    npad_l = _KS // 2
    nch = (T + 2 * npad_l + _CH - 1) // _CH

    bs = pl.BlockSpec((1, C, T), lambda i: (i, 0, 0))

    trend, s_e = pl.pallas_call(
        _decomp_kernel,
        grid=(B,),
        in_specs=[bs],
        out_specs=[bs, bs],
        out_shape=[jax.ShapeDtypeStruct((B, C, T), jnp.float32)] * 2,
        scratch_shapes=[pltpu.VMEM((_CH, nch * C), jnp.float32)],
    )(x)

    fx = jnp.fft.rfft(s_e, axis=-1)
    ac = jnp.fft.irfft(fx * jnp.conj(fx), n=T, axis=-1)

    out, agg = pl.pallas_call(
        _agg_kernel,
        grid=(B,),
        in_specs=[
            bs, bs, bs,
            pl.BlockSpec((C, C), lambda i: (0, 0)),
            pl.BlockSpec((C, 1), lambda i: (0, 0)),
        ],
        out_specs=[bs, bs],
        out_shape=[jax.ShapeDtypeStruct((B, C, T), jnp.float32)] * 2,
    )(ac, s_e, trend, W, b.reshape(C, 1))
    return out, trend, agg


# speedup vs baseline: 4.8623x; 1.0148x over previous
import jax
import jax.numpy as jnp
from jax.experimental import pallas as pl
from jax.experimental.pallas import tpu as pltpu

_KS = 25
_NBITS = 12
_CH = 128


def _decomp_kernel(x_ref, cs_ref, scan_ref):
    x = x_ref[0]
    C, T = x.shape
    npad_l = _KS // 2
    n_in = T + 2 * npad_l
    nch = (n_in + _CH - 1) // _CH
    npad_r = nch * _CH - T - npad_l

    zl = jnp.zeros((C, npad_l), jnp.float32)
    zr = jnp.zeros((C, npad_r), jnp.float32)
    xp = jnp.concatenate([zl, x, zr], axis=1)

    cols = [xp[:, c * _CH:(c + 1) * _CH].T for c in range(nch)]
    scan_ref[...] = jnp.concatenate(cols, axis=1)

    def body(r, carry):
        nxt = carry + scan_ref[pl.ds(r, 1), :]
        scan_ref[pl.ds(r, 1), :] = nxt
        return nxt

    totals = jax.lax.fori_loop(1, _CH, body, scan_ref[pl.ds(0, 1), :])

    offs = [jnp.zeros((1, C), jnp.float32)]
    for c in range(1, nch):
        offs.append(offs[-1] + totals[:, (c - 1) * C:c * C])

    cs_cols = []
    for c in range(nch):
        blk = scan_ref[:, c * C:(c + 1) * C] + offs[c]
        cs_cols.append(blk.T)
    cs = jnp.concatenate(cs_cols, axis=1)

    cs_ref[0] = cs


def _first_argmax(a, fill):
    m = jnp.max(a, axis=-1, keepdims=True)
    idx = jax.lax.broadcasted_iota(jnp.int32, a.shape, 1)
    only = jnp.where(a == m, idx, fill)
    return jnp.min(only, axis=-1, keepdims=True)


def _agg_kernel(ac_ref, s_ref, trend_ref, w_ref, b_ref, out_ref, agg_ref):
    ac = ac_ref[0]
    s = s_ref[0]
    trend = trend_ref[0]
    C, T = s.shape

    lane = jax.lax.broadcasted_iota(jnp.int32, ac.shape, 1)
    neginf = jnp.float32(-jnp.inf)
    cand = jnp.where(lane == 0, neginf, ac)
    m1 = _first_argmax(cand, T)
    cand = jnp.where(lane == m1, neginf, cand)
    m2 = _first_argmax(cand, T)

    def shifted(L):
        y = s
        for bit in range(_NBITS):
            amt = 1 << bit
            rolled = jnp.concatenate([y[:, T - amt:], y[:, : T - amt]], axis=1)
            y = jnp.where(((L >> bit) & 1) == 1, rolled, y)
        return jnp.where(lane >= L, y, 0.0)

    agg = (s + shifted(m1) + shifted(m2)) * (1.0 / 3.0)

    out = jax.lax.dot_general(
        w_ref[...], agg + trend, (((1,), (0,)), ((), ())),
        preferred_element_type=jnp.float32,
        precision=jax.lax.Precision.HIGHEST) + b_ref[...]

    out_ref[0] = out
    agg_ref[0] = agg


def kernel(x, W, b):
    B, C, T = x.shape
    npad_l = _KS // 2
    n_in = T + 2 * npad_l
    nch = (n_in + _CH - 1) // _CH

    bs = pl.BlockSpec((1, C, T), lambda i: (i, 0, 0))

    cs_full = pl.pallas_call(
        _decomp_kernel,
        grid=(B,),
        in_specs=[bs],
        out_specs=pl.BlockSpec((1, C, nch * _CH), lambda i: (i, 0, 0)),
        out_shape=jax.ShapeDtypeStruct((B, C, nch * _CH), jnp.float32),
        scratch_shapes=[pltpu.VMEM((_CH, nch * C), jnp.float32)],
    )(x)

    cs = cs_full[..., :n_in]
    zero = jnp.zeros(cs.shape[:-1] + (1,), cs.dtype)
    cs = jnp.concatenate([zero, cs], axis=-1)
    trend = (cs[..., _KS:] - cs[..., :-_KS]) / _KS
    s_e = x - trend
    fx = jnp.fft.rfft(s_e, axis=-1)
    ac = jnp.fft.irfft(fx * jnp.conj(fx), n=T, axis=-1)

    out, agg = pl.pallas_call(
        _agg_kernel,
        grid=(B,),
        in_specs=[
            bs, bs, bs,
            pl.BlockSpec((C, C), lambda i: (0, 0)),
            pl.BlockSpec((C, 1), lambda i: (0, 0)),
        ],
        out_specs=[bs, bs],
        out_shape=[jax.ShapeDtypeStruct((B, C, T), jnp.float32)] * 2,
    )(ac, s_e, trend, W, b.reshape(C, 1))
    return out, trend, agg
